# baseline (speedup 1.0000x reference)
import functools

import jax
import jax.numpy as jnp
from jax import lax
from jax.experimental import pallas as pl
from jax.experimental.pallas import tpu as pltpu, tpu_sc as plsc

N = 10000
D = 128
H = 64
G = 128

NC = 2
NS = 16
CHUNK = 128
VROWS = 10240
PT = VROWS // NS
TRASH = N


def _sc_mesh():
    return plsc.VectorSubcoreMesh(core_axis_name="c", subcore_axis_name="s")


_SC_PARAMS = pltpu.CompilerParams(use_tc_tiling_on_sc=False)


def _sc_degree(dst4):
    nch = dst4.shape[2]

    @functools.partial(
        pl.kernel,
        out_type=jax.ShapeDtypeStruct((NC, VROWS), jnp.float32),
        mesh=_sc_mesh(),
        scratch_types=[
            pltpu.VMEM((nch, CHUNK), jnp.int32),
            pltpu.VMEM((CHUNK,), jnp.float32),
            pltpu.VMEM((PT,), jnp.float32),
            pltpu.MemorySpace.VMEM_SHARED((VROWS,), jnp.float32),
        ],
        compiler_params=_SC_PARAMS,
    )
    def deg_kernel(dst_hbm, out_hbm, didx, ones_v, zbuf, acc):
        c = lax.axis_index("c")
        s = lax.axis_index("s")

        def fill1(i, _):
            ones_v[pl.ds(i * 16, 16)] = jnp.ones((16,), jnp.float32)
            return 0

        def fill0(i, _):
            zbuf[pl.ds(i * 16, 16)] = jnp.zeros((16,), jnp.float32)
            return 0

        lax.fori_loop(0, CHUNK // 16, fill1, 0)
        lax.fori_loop(0, PT // 16, fill0, 0)
        pltpu.sync_copy(zbuf, acc.at[pl.ds(s * PT, PT)])
        pltpu.sync_copy(dst_hbm.at[c, s], didx)
        plsc.subcore_barrier()

        def chunk(j, _):
            pltpu.sync_copy(ones_v, acc.at[didx.at[j]], add=True)
            return 0

        lax.fori_loop(0, nch, chunk, 0)
        plsc.subcore_barrier()
        pltpu.sync_copy(acc.at[pl.ds(s * PT, PT)],
                        out_hbm.at[c, pl.ds(s * PT, PT)])

    return deg_kernel(dst4)


def _sc_edge_pass(src4, dst4, table):
    nch = src4.shape[2]

    @functools.partial(
        pl.kernel,
        out_type=jax.ShapeDtypeStruct((NC, VROWS, H), jnp.float32),
        mesh=_sc_mesh(),
        scratch_types=[
            pltpu.VMEM((nch, CHUNK), jnp.int32),
            pltpu.VMEM((nch, CHUNK), jnp.int32),
        ] + [pltpu.VMEM((CHUNK, H), jnp.float32)] * 4 + [
            pltpu.MemorySpace.VMEM_SHARED((VROWS, H), jnp.float32),
        ] + [pltpu.SemaphoreType.DMA] * 8,
        compiler_params=_SC_PARAMS,
    )
    def pass_kernel(src_hbm, dst_hbm, tab_hbm, out_hbm, sidx, didx, *rest):
        bufs = rest[0:4]
        acc = rest[4]
        gsems = rest[5:9]
        ssems = rest[9:13]
        c = lax.axis_index("c")
        s = lax.axis_index("s")
        rows0 = bufs[0]

        def zrow(i, _):
            for j in range(H // 16):
                rows0[i, pl.ds(j * 16, 16)] = jnp.zeros((16,), jnp.float32)
            return 0

        lax.fori_loop(0, CHUNK, zrow, 0)

        def zc(k, _):
            pltpu.sync_copy(rows0, acc.at[pl.ds(s * PT + k * CHUNK, CHUNK), :])
            return 0

        lax.fori_loop(0, PT // CHUNK, zc, 0)
        pltpu.sync_copy(src_hbm.at[c, s], sidx)
        pltpu.sync_copy(dst_hbm.at[c, s], didx)
        plsc.subcore_barrier()

        for b in range(2):
            pltpu.async_copy(tab_hbm.at[sidx.at[b]], bufs[b], gsems[b])

        def chunk(j, _):
            for b in range(4):
                @pl.when(lax.rem(j, 4) == b)
                def _():
                    b2 = (b + 2) % 4
                    pltpu.make_async_copy(tab_hbm.at[sidx.at[j]],
                                          bufs[b], gsems[b]).wait()
                    pltpu.async_copy(bufs[b], acc.at[didx.at[j]], ssems[b],
                                     add=True)

                    @pl.when(j >= 2)
                    def _():
                        pltpu.make_async_copy(
                            bufs[b2], acc.at[didx.at[j - 2]],
                            ssems[b2]).wait()

                    @pl.when(j + 2 < nch)
                    def _():
                        pltpu.async_copy(tab_hbm.at[sidx.at[j + 2]],
                                         bufs[b2], gsems[b2])
            return 0

        lax.fori_loop(0, nch, chunk, 0)
        for k in (nch - 2, nch - 1):
            b = k % 4
            pltpu.make_async_copy(bufs[b], acc.at[didx.at[k]],
                                  ssems[b]).wait()
        plsc.subcore_barrier()
        pltpu.sync_copy(acc.at[pl.ds(s * PT, PT), :],
                        out_hbm.at[c, pl.ds(s * PT, PT), :])

    return pass_kernel(src4, dst4, table)


BR = 1000
NBV = N // BR


def _ka_body(x_ref, fm_ref, w1_ref, deg_ref, y_ref, dinv_ref):
    v = pl.program_id(0)
    xb = x_ref[...]
    xb = jnp.where(v == 1, xb * fm_ref[...], xb)
    deg = deg_ref[...] + 1.0
    dinv = lax.rsqrt(deg)
    y = jnp.dot(xb, w1_ref[...], preferred_element_type=jnp.float32)
    y_ref[...] = y * dinv
    dinv_ref[...] = dinv


def _tc_ka(x, fmask, W1, deg):
    return pl.pallas_call(
        _ka_body,
        grid=(2, NBV),
        in_specs=[
            pl.BlockSpec((BR, D), lambda v, i: (i, 0)),
            pl.BlockSpec((1, D), lambda v, i: (0, 0)),
            pl.BlockSpec((D, H), lambda v, i: (0, 0)),
            pl.BlockSpec((BR, 1), lambda v, i: (v * NBV + i, 0)),
        ],
        out_specs=[
            pl.BlockSpec((BR, H), lambda v, i: (v * NBV + i, 0)),
            pl.BlockSpec((BR, 1), lambda v, i: (v * NBV + i, 0)),
        ],
        out_shape=[
            jax.ShapeDtypeStruct((2 * N, H), jnp.float32),
            jax.ShapeDtypeStruct((2 * N, 1), jnp.float32),
        ],
    )(x, fmask, W1, deg)


def _kb_body(acc_ref, y_ref, dinv_ref, b1_ref, w2_ref, out_ref):
    d = dinv_ref[...]
    h = jnp.maximum(d * acc_ref[...] + d * y_ref[...] + b1_ref[...], 0.0)
    out_ref[...] = jnp.dot(h, w2_ref[...],
                           preferred_element_type=jnp.float32) * d


def _tc_kb(acc, y, dinv, b1, W2):
    return pl.pallas_call(
        _kb_body,
        grid=(2 * NBV,),
        in_specs=[
            pl.BlockSpec((BR, H), lambda i: (i, 0)),
            pl.BlockSpec((BR, H), lambda i: (i, 0)),
            pl.BlockSpec((BR, 1), lambda i: (i, 0)),
            pl.BlockSpec((1, H), lambda i: (0, 0)),
            pl.BlockSpec((H, H), lambda i: (0, 0)),
        ],
        out_specs=pl.BlockSpec((BR, H), lambda i: (i, 0)),
        out_shape=jax.ShapeDtypeStruct((2 * N, H), jnp.float32),
    )(acc, y, dinv, b1, W2)


def _kc_body(acc_ref, t_ref, dinv_ref, b2_ref, bat_ref, out_ref):
    j = pl.program_id(1)
    d = dinv_ref[...]
    o2 = d * acc_ref[...] + d * t_ref[...] + b2_ref[...]
    oh = (bat_ref[...] == lax.broadcasted_iota(jnp.int32, (1, G), 1)
          ).astype(jnp.float32)
    zp = lax.dot_general(oh, o2, (((0,), (0,)), ((), ())),
                         preferred_element_type=jnp.float32)

    @pl.when(j == 0)
    def _():
        out_ref[...] = zp[None]

    @pl.when(j > 0)
    def _():
        out_ref[...] += zp[None]


def _tc_kc(acc, t, dinv, b2, batc):
    return pl.pallas_call(
        _kc_body,
        grid=(2, NBV),
        in_specs=[
            pl.BlockSpec((BR, H), lambda v, i: (v * NBV + i, 0)),
            pl.BlockSpec((BR, H), lambda v, i: (v * NBV + i, 0)),
            pl.BlockSpec((BR, 1), lambda v, i: (v * NBV + i, 0)),
            pl.BlockSpec((1, H), lambda v, i: (0, 0)),
            pl.BlockSpec((BR, 1), lambda v, i: (v * NBV + i, 0)),
        ],
        out_specs=pl.BlockSpec((1, G, H), lambda v, i: (v, 0, 0)),
        out_shape=jax.ShapeDtypeStruct((2, G, H), jnp.float32),
        compiler_params=pltpu.CompilerParams(
            dimension_semantics=("arbitrary", "arbitrary")),
    )(acc, t, dinv, b2, batc)


def kernel(x, edge_index, batch, W1, b1, W2, b2):
    E = edge_index.shape[1]
    src, dst = edge_index[0], edge_index[1]

    akey = jax.random.key(42)
    ka, kb = jax.random.split(akey)
    keep = jax.random.uniform(ka, (E,)) >= 0.1
    fmask = (jax.random.uniform(kb, (D,)) >= 0.1).astype(jnp.float32)

    per_tile = -(-E // (NS * CHUNK * 16)) * CHUNK * 16
    pad = NS * per_tile - E
    trash = TRASH + lax.rem(jnp.arange(E, dtype=jnp.int32), 240)
    padi = jnp.zeros((pad,), jnp.int32)
    padt = TRASH + lax.rem(jnp.arange(pad, dtype=jnp.int32), 240)
    src4 = jnp.stack([jnp.concatenate([src, padi]),
                      jnp.concatenate([src + N, padi])])
    dst4 = jnp.stack([jnp.concatenate([jnp.where(keep, dst, trash), padt]),
                      jnp.concatenate([dst, padt])])
    src4 = src4.reshape(NC, NS, per_tile // CHUNK, CHUNK)
    dst4 = dst4.reshape(NC, NS, per_tile // CHUNK, CHUNK)

    degp = _sc_degree(dst4)
    deg2d = degp[:, :N].reshape(2 * N, 1)

    ycat, dinv = _tc_ka(x, fmask[None, :], W1, deg2d)
    acc1 = _sc_edge_pass(src4, dst4, ycat)
    acc1v = acc1[:, :N, :].reshape(2 * N, H)
    tcat = _tc_kb(acc1v, ycat, dinv, b1[None, :], W2)
    acc2 = _sc_edge_pass(src4, dst4, tcat)
    acc2v = acc2[:, :N, :].reshape(2 * N, H)
    batc = jnp.concatenate([batch, batch])[:, None]
    zcat = _tc_kc(acc2v, tcat, dinv, b2[None, :], batc)
    return (zcat[0], zcat[1])


# speedup vs baseline: 1.9267x; 1.9267x over previous
import functools

import jax
import jax.numpy as jnp
from jax import lax
from jax.experimental import pallas as pl
from jax.experimental.pallas import tpu as pltpu, tpu_sc as plsc

N = 10000
D = 128
H = 64
G = 128

NC = 2
NS = 16
CHUNK = 128
VROWS = 10240
PT = VROWS // NS
TRASH = N


def _sc_mesh():
    return plsc.VectorSubcoreMesh(core_axis_name="c", subcore_axis_name="s")


_SC_PARAMS = pltpu.CompilerParams(use_tc_tiling_on_sc=False)


def _sc_degree(dst4):
    nch = dst4.shape[2]

    @functools.partial(
        pl.kernel,
        out_type=jax.ShapeDtypeStruct((NC, VROWS), jnp.float32),
        mesh=_sc_mesh(),
        scratch_types=[
            pltpu.VMEM((nch, CHUNK), jnp.int32),
            pltpu.VMEM((CHUNK,), jnp.float32),
            pltpu.VMEM((PT,), jnp.float32),
            pltpu.MemorySpace.VMEM_SHARED((VROWS,), jnp.float32),
        ],
        compiler_params=_SC_PARAMS,
    )
    def deg_kernel(dst_hbm, out_hbm, didx, ones_v, zbuf, acc):
        c = lax.axis_index("c")
        s = lax.axis_index("s")

        def fill1(i, _):
            ones_v[pl.ds(i * 16, 16)] = jnp.ones((16,), jnp.float32)
            return 0

        def fill0(i, _):
            zbuf[pl.ds(i * 16, 16)] = jnp.zeros((16,), jnp.float32)
            return 0

        lax.fori_loop(0, CHUNK // 16, fill1, 0)
        lax.fori_loop(0, PT // 16, fill0, 0)
        pltpu.sync_copy(zbuf, acc.at[pl.ds(s * PT, PT)])
        pltpu.sync_copy(dst_hbm.at[c, s], didx)
        plsc.subcore_barrier()

        def chunk(j, _):
            pltpu.sync_copy(ones_v, acc.at[didx.at[j]], add=True)
            return 0

        lax.fori_loop(0, nch, chunk, 0)
        plsc.subcore_barrier()
        pltpu.sync_copy(acc.at[pl.ds(s * PT, PT)],
                        out_hbm.at[c, pl.ds(s * PT, PT)])

    return deg_kernel(dst4)


def _sc_edge_pass(src4, dst4, table):
    nch = src4.shape[2]

    @functools.partial(
        pl.kernel,
        out_type=jax.ShapeDtypeStruct((NC, VROWS, H), jnp.float32),
        mesh=_sc_mesh(),
        scratch_types=[
            pltpu.VMEM((nch, CHUNK), jnp.int32),
            pltpu.VMEM((nch, CHUNK), jnp.int32),
        ] + [pltpu.VMEM((CHUNK, H), jnp.float32)] * 6 + [
            pltpu.MemorySpace.VMEM_SHARED((VROWS, H), jnp.float32),
        ] + [pltpu.SemaphoreType.DMA] * 12,
        compiler_params=_SC_PARAMS,
    )
    def pass_kernel(src_hbm, dst_hbm, tab_hbm, out_hbm, sidx, didx, *rest):
        bufs = rest[0:6]
        acc = rest[6]
        gsems = rest[7:13]
        ssems = rest[13:19]
        c = lax.axis_index("c")
        s = lax.axis_index("s")
        rows0 = bufs[0]

        def zrow(i, _):
            for j in range(H // 16):
                rows0[i, pl.ds(j * 16, 16)] = jnp.zeros((16,), jnp.float32)
            return 0

        lax.fori_loop(0, CHUNK, zrow, 0)

        def zc(k, _):
            pltpu.sync_copy(rows0, acc.at[pl.ds(s * PT + k * CHUNK, CHUNK), :])
            return 0

        lax.fori_loop(0, PT // CHUNK, zc, 0)
        pltpu.sync_copy(src_hbm.at[c, s], sidx)
        pltpu.sync_copy(dst_hbm.at[c, s], didx)
        plsc.subcore_barrier()

        for b in range(3):
            pltpu.async_copy(tab_hbm.at[sidx.at[b]], bufs[b], gsems[b])

        def chunk(j, _):
            for b in range(6):
                @pl.when(lax.rem(j, 6) == b)
                def _():
                    b2 = (b + 3) % 6
                    pltpu.make_async_copy(tab_hbm.at[sidx.at[j]],
                                          bufs[b], gsems[b]).wait()
                    pltpu.async_copy(bufs[b], acc.at[didx.at[j]], ssems[b],
                                     add=True)

                    @pl.when(j >= 3)
                    def _():
                        pltpu.make_async_copy(
                            bufs[b2], acc.at[didx.at[j - 3]],
                            ssems[b2]).wait()

                    @pl.when(j + 3 < nch)
                    def _():
                        pltpu.async_copy(tab_hbm.at[sidx.at[j + 3]],
                                         bufs[b2], gsems[b2])
            return 0

        lax.fori_loop(0, nch, chunk, 0)
        for k in (nch - 3, nch - 2, nch - 1):
            b = k % 6
            pltpu.make_async_copy(bufs[b], acc.at[didx.at[k]],
                                  ssems[b]).wait()
        plsc.subcore_barrier()
        pltpu.sync_copy(acc.at[pl.ds(s * PT, PT), :],
                        out_hbm.at[c, pl.ds(s * PT, PT), :])

    return pass_kernel(src4, dst4, table)


BR = 1000
NBV = N // BR


def _ka_body(x_ref, fm_ref, w1_ref, deg_ref, y_ref, dinv_ref):
    v = pl.program_id(0)
    xb = x_ref[...]
    xb = jnp.where(v == 1, xb * fm_ref[...], xb)
    deg = deg_ref[...] + 1.0
    dinv = lax.rsqrt(deg)
    y = jnp.dot(xb, w1_ref[...], preferred_element_type=jnp.float32)
    y_ref[...] = y * dinv
    dinv_ref[...] = dinv


def _tc_ka(x, fmask, W1, deg):
    return pl.pallas_call(
        _ka_body,
        grid=(2, NBV),
        in_specs=[
            pl.BlockSpec((BR, D), lambda v, i: (i, 0)),
            pl.BlockSpec((1, D), lambda v, i: (0, 0)),
            pl.BlockSpec((D, H), lambda v, i: (0, 0)),
            pl.BlockSpec((BR, 1), lambda v, i: (v * NBV + i, 0)),
        ],
        out_specs=[
            pl.BlockSpec((BR, H), lambda v, i: (v * NBV + i, 0)),
            pl.BlockSpec((BR, 1), lambda v, i: (v * NBV + i, 0)),
        ],
        out_shape=[
            jax.ShapeDtypeStruct((2 * N, H), jnp.float32),
            jax.ShapeDtypeStruct((2 * N, 1), jnp.float32),
        ],
    )(x, fmask, W1, deg)


def _kb_body(acc_ref, y_ref, dinv_ref, b1_ref, w2_ref, out_ref):
    d = dinv_ref[...]
    h = jnp.maximum(d * acc_ref[...] + d * y_ref[...] + b1_ref[...], 0.0)
    out_ref[...] = jnp.dot(h, w2_ref[...],
                           preferred_element_type=jnp.float32) * d


def _tc_kb(acc, y, dinv, b1, W2):
    return pl.pallas_call(
        _kb_body,
        grid=(2 * NBV,),
        in_specs=[
            pl.BlockSpec((BR, H), lambda i: (i, 0)),
            pl.BlockSpec((BR, H), lambda i: (i, 0)),
            pl.BlockSpec((BR, 1), lambda i: (i, 0)),
            pl.BlockSpec((1, H), lambda i: (0, 0)),
            pl.BlockSpec((H, H), lambda i: (0, 0)),
        ],
        out_specs=pl.BlockSpec((BR, H), lambda i: (i, 0)),
        out_shape=jax.ShapeDtypeStruct((2 * N, H), jnp.float32),
    )(acc, y, dinv, b1, W2)


def _kc_body(acc_ref, t_ref, dinv_ref, b2_ref, bat_ref, out_ref):
    j = pl.program_id(1)
    d = dinv_ref[...]
    o2 = d * acc_ref[...] + d * t_ref[...] + b2_ref[...]
    oh = (bat_ref[...] == lax.broadcasted_iota(jnp.int32, (1, G), 1)
          ).astype(jnp.float32)
    zp = lax.dot_general(oh, o2, (((0,), (0,)), ((), ())),
                         preferred_element_type=jnp.float32)

    @pl.when(j == 0)
    def _():
        out_ref[...] = zp[None]

    @pl.when(j > 0)
    def _():
        out_ref[...] += zp[None]


def _tc_kc(acc, t, dinv, b2, batc):
    return pl.pallas_call(
        _kc_body,
        grid=(2, NBV),
        in_specs=[
            pl.BlockSpec((BR, H), lambda v, i: (v * NBV + i, 0)),
            pl.BlockSpec((BR, H), lambda v, i: (v * NBV + i, 0)),
            pl.BlockSpec((BR, 1), lambda v, i: (v * NBV + i, 0)),
            pl.BlockSpec((1, H), lambda v, i: (0, 0)),
            pl.BlockSpec((BR, 1), lambda v, i: (v * NBV + i, 0)),
        ],
        out_specs=pl.BlockSpec((1, G, H), lambda v, i: (v, 0, 0)),
        out_shape=jax.ShapeDtypeStruct((2, G, H), jnp.float32),
        compiler_params=pltpu.CompilerParams(
            dimension_semantics=("arbitrary", "arbitrary")),
    )(acc, t, dinv, b2, batc)


def kernel(x, edge_index, batch, W1, b1, W2, b2):
    E = edge_index.shape[1]
    src, dst = edge_index[0], edge_index[1]

    akey = jax.random.key(42)
    ka, kb = jax.random.split(akey)
    keep = jax.random.uniform(ka, (E,)) >= 0.1
    fmask = (jax.random.uniform(kb, (D,)) >= 0.1).astype(jnp.float32)

    per_tile = -(-E // (NS * CHUNK)) * CHUNK
    pad = NS * per_tile - E
    trash = TRASH + lax.rem(jnp.arange(E, dtype=jnp.int32), 240)
    padi = jnp.zeros((pad,), jnp.int32)
    padt = TRASH + lax.rem(jnp.arange(pad, dtype=jnp.int32), 240)
    src4 = jnp.stack([jnp.concatenate([src, padi]),
                      jnp.concatenate([src + N, padi])])
    dst4 = jnp.stack([jnp.concatenate([jnp.where(keep, dst, trash), padt]),
                      jnp.concatenate([dst, padt])])
    src4 = src4.reshape(NC, NS, per_tile // CHUNK, CHUNK)
    dst4 = dst4.reshape(NC, NS, per_tile // CHUNK, CHUNK)

    degp = _sc_degree(dst4)
    deg2d = degp[:, :N].reshape(2 * N, 1)

    ycat, dinv = _tc_ka(x, fmask[None, :], W1, deg2d)
    acc1 = _sc_edge_pass(src4, dst4, ycat)
    acc1v = acc1[:, :N, :].reshape(2 * N, H)
    tcat = _tc_kb(acc1v, ycat, dinv, b1[None, :], W2)
    acc2 = _sc_edge_pass(src4, dst4, tcat)
    acc2v = acc2[:, :N, :].reshape(2 * N, H)
    batc = jnp.concatenate([batch, batch])[:, None]
    zcat = _tc_kc(acc2v, tcat, dinv, b2[None, :], batc)
    return (zcat[0], zcat[1])


# speedup vs baseline: 1.9274x; 1.0004x over previous
import functools

import jax
import jax.numpy as jnp
from jax import lax
from jax.experimental import pallas as pl
from jax.experimental.pallas import tpu as pltpu, tpu_sc as plsc

N = 10000
D = 128
H = 64
G = 128

NC = 2
NS = 16
CHUNK = 128
VROWS = 10240
PT = VROWS // NS
TRASH = N


def _sc_mesh():
    return plsc.VectorSubcoreMesh(core_axis_name="c", subcore_axis_name="s")


_SC_PARAMS = pltpu.CompilerParams(use_tc_tiling_on_sc=False)


def _sc_degree(dst4):
    nch = dst4.shape[2]

    @functools.partial(
        pl.kernel,
        out_type=jax.ShapeDtypeStruct((NC, VROWS), jnp.float32),
        mesh=_sc_mesh(),
        scratch_types=[
            pltpu.VMEM((nch, CHUNK), jnp.int32),
            pltpu.VMEM((CHUNK,), jnp.float32),
            pltpu.VMEM((PT,), jnp.float32),
            pltpu.MemorySpace.VMEM_SHARED((VROWS,), jnp.float32),
        ],
        compiler_params=_SC_PARAMS,
    )
    def deg_kernel(dst_hbm, out_hbm, didx, ones_v, zbuf, acc):
        c = lax.axis_index("c")
        s = lax.axis_index("s")

        def fill1(i, _):
            ones_v[pl.ds(i * 16, 16)] = jnp.ones((16,), jnp.float32)
            return 0

        def fill0(i, _):
            zbuf[pl.ds(i * 16, 16)] = jnp.zeros((16,), jnp.float32)
            return 0

        lax.fori_loop(0, CHUNK // 16, fill1, 0)
        lax.fori_loop(0, PT // 16, fill0, 0)
        pltpu.sync_copy(zbuf, acc.at[pl.ds(s * PT, PT)])
        pltpu.sync_copy(dst_hbm.at[c, s], didx)
        plsc.subcore_barrier()

        def chunk(j, _):
            pltpu.sync_copy(ones_v, acc.at[didx.at[j]], add=True)
            return 0

        lax.fori_loop(0, nch, chunk, 0)
        plsc.subcore_barrier()
        pltpu.sync_copy(acc.at[pl.ds(s * PT, PT)],
                        out_hbm.at[c, pl.ds(s * PT, PT)])

    return deg_kernel(dst4)


def _sc_edge_pass(src4, dst4, table):
    nch = src4.shape[2]

    @functools.partial(
        pl.kernel,
        out_type=jax.ShapeDtypeStruct((NC, VROWS, H), jnp.float32),
        mesh=_sc_mesh(),
        scratch_types=[
            pltpu.VMEM((nch, CHUNK), jnp.int32),
            pltpu.VMEM((nch, CHUNK), jnp.int32),
        ] + [pltpu.VMEM((CHUNK, H), jnp.float32)] * 6 + [
            pltpu.VMEM((CHUNK,), jnp.float32),
            pltpu.MemorySpace.VMEM_SHARED((VROWS, H), jnp.float32),
        ] + [pltpu.SemaphoreType.DMA] * 12,
        compiler_params=_SC_PARAMS,
    )
    def pass_kernel(src_hbm, dst_hbm, tab_hbm, out_hbm, sidx, didx, *rest):
        bufs = rest[0:6]
        acc = rest[7]
        gsems = rest[8:14]
        ssems = rest[14:20]
        c = lax.axis_index("c")
        s = lax.axis_index("s")
        rows0 = bufs[0]

        def zrow(i, _):
            for j in range(H // 16):
                rows0[i, pl.ds(j * 16, 16)] = jnp.zeros((16,), jnp.float32)
            return 0

        lax.fori_loop(0, CHUNK, zrow, 0)

        def zc(k, _):
            pltpu.sync_copy(rows0, acc.at[pl.ds(s * PT + k * CHUNK, CHUNK), :])
            return 0

        lax.fori_loop(0, PT // CHUNK, zc, 0)
        pltpu.sync_copy(src_hbm.at[c, s], sidx)
        pltpu.sync_copy(dst_hbm.at[c, s], didx)
        plsc.subcore_barrier()

        for b in range(3):
            pltpu.async_copy(tab_hbm.at[sidx.at[b]], bufs[b], gsems[b])

        def chunk(j, _):
            for b in range(6):
                @pl.when(lax.rem(j, 6) == b)
                def _():
                    b2 = (b + 3) % 6
                    pltpu.make_async_copy(tab_hbm.at[sidx.at[j]],
                                          bufs[b], gsems[b]).wait()
                    pltpu.async_copy(bufs[b], acc.at[didx.at[j]], ssems[b],
                                     add=True)

                    @pl.when(j >= 3)
                    def _():
                        pltpu.make_async_copy(
                            bufs[b2], acc.at[didx.at[j - 3]],
                            ssems[b2]).wait()

                    @pl.when(j + 3 < nch)
                    def _():
                        pltpu.async_copy(tab_hbm.at[sidx.at[j + 3]],
                                         bufs[b2], gsems[b2])
            return 0

        lax.fori_loop(0, nch, chunk, 0)
        for k in (nch - 3, nch - 2, nch - 1):
            b = k % 6
            pltpu.make_async_copy(bufs[b], acc.at[didx.at[k]],
                                  ssems[b]).wait()
        plsc.subcore_barrier()
        pltpu.sync_copy(acc.at[pl.ds(s * PT, PT), :],
                        out_hbm.at[c, pl.ds(s * PT, PT), :])

    return pass_kernel(src4, dst4, table)


BR = 1000
NBV = N // BR


def _ka_body(x_ref, fm_ref, w1_ref, deg_ref, y_ref, dinv_ref):
    v = pl.program_id(0)
    xb = x_ref[...]
    xb = jnp.where(v == 1, xb * fm_ref[...], xb)
    deg = deg_ref[...] + 1.0
    dinv = lax.rsqrt(deg)
    y = jnp.dot(xb, w1_ref[...], preferred_element_type=jnp.float32)
    y_ref[...] = y * dinv
    dinv_ref[...] = dinv


def _tc_ka(x, fmask, W1, deg):
    return pl.pallas_call(
        _ka_body,
        grid=(2, NBV),
        in_specs=[
            pl.BlockSpec((BR, D), lambda v, i: (i, 0)),
            pl.BlockSpec((1, D), lambda v, i: (0, 0)),
            pl.BlockSpec((D, H), lambda v, i: (0, 0)),
            pl.BlockSpec((BR, 1), lambda v, i: (v * NBV + i, 0)),
        ],
        out_specs=[
            pl.BlockSpec((BR, H), lambda v, i: (v * NBV + i, 0)),
            pl.BlockSpec((BR, 1), lambda v, i: (v * NBV + i, 0)),
        ],
        out_shape=[
            jax.ShapeDtypeStruct((2 * N, H), jnp.float32),
            jax.ShapeDtypeStruct((2 * N, 1), jnp.float32),
        ],
    )(x, fmask, W1, deg)


def _kb_body(acc_ref, y_ref, dinv_ref, b1_ref, w2_ref, out_ref):
    d = dinv_ref[...]
    h = jnp.maximum(d * acc_ref[...] + d * y_ref[...] + b1_ref[...], 0.0)
    out_ref[...] = jnp.dot(h, w2_ref[...],
                           preferred_element_type=jnp.float32) * d


def _tc_kb(acc, y, dinv, b1, W2):
    return pl.pallas_call(
        _kb_body,
        grid=(2 * NBV,),
        in_specs=[
            pl.BlockSpec((BR, H), lambda i: (i, 0)),
            pl.BlockSpec((BR, H), lambda i: (i, 0)),
            pl.BlockSpec((BR, 1), lambda i: (i, 0)),
            pl.BlockSpec((1, H), lambda i: (0, 0)),
            pl.BlockSpec((H, H), lambda i: (0, 0)),
        ],
        out_specs=pl.BlockSpec((BR, H), lambda i: (i, 0)),
        out_shape=jax.ShapeDtypeStruct((2 * N, H), jnp.float32),
    )(acc, y, dinv, b1, W2)


def _kc_body(acc_ref, t_ref, dinv_ref, b2_ref, bat_ref, out_ref):
    j = pl.program_id(1)
    d = dinv_ref[...]
    o2 = d * acc_ref[...] + d * t_ref[...] + b2_ref[...]
    oh = (bat_ref[...] == lax.broadcasted_iota(jnp.int32, (1, G), 1)
          ).astype(jnp.float32)
    zp = lax.dot_general(oh, o2, (((0,), (0,)), ((), ())),
                         preferred_element_type=jnp.float32)

    @pl.when(j == 0)
    def _():
        out_ref[...] = zp[None]

    @pl.when(j > 0)
    def _():
        out_ref[...] += zp[None]


def _tc_kc(acc, t, dinv, b2, batc):
    return pl.pallas_call(
        _kc_body,
        grid=(2, NBV),
        in_specs=[
            pl.BlockSpec((BR, H), lambda v, i: (v * NBV + i, 0)),
            pl.BlockSpec((BR, H), lambda v, i: (v * NBV + i, 0)),
            pl.BlockSpec((BR, 1), lambda v, i: (v * NBV + i, 0)),
            pl.BlockSpec((1, H), lambda v, i: (0, 0)),
            pl.BlockSpec((BR, 1), lambda v, i: (v * NBV + i, 0)),
        ],
        out_specs=pl.BlockSpec((1, G, H), lambda v, i: (v, 0, 0)),
        out_shape=jax.ShapeDtypeStruct((2, G, H), jnp.float32),
        compiler_params=pltpu.CompilerParams(
            dimension_semantics=("arbitrary", "arbitrary")),
    )(acc, t, dinv, b2, batc)


def kernel(x, edge_index, batch, W1, b1, W2, b2):
    E = edge_index.shape[1]
    src, dst = edge_index[0], edge_index[1]

    akey = jax.random.key(42)
    ka, kb = jax.random.split(akey)
    keep = jax.random.uniform(ka, (E,)) >= 0.1
    fmask = (jax.random.uniform(kb, (D,)) >= 0.1).astype(jnp.float32)

    per_tile = -(-E // (NS * CHUNK)) * CHUNK
    pad = NS * per_tile - E
    trash = TRASH + lax.rem(jnp.arange(E, dtype=jnp.int32), 240)
    padi = jnp.zeros((pad,), jnp.int32)
    padt = TRASH + lax.rem(jnp.arange(pad, dtype=jnp.int32), 240)
    src4 = jnp.stack([jnp.concatenate([src, padi]),
                      jnp.concatenate([src + N, padi])])
    dst4 = jnp.stack([jnp.concatenate([jnp.where(keep, dst, trash), padt]),
                      jnp.concatenate([dst, padt])])
    src4 = src4.reshape(NC, NS, per_tile // CHUNK, CHUNK)
    dst4 = dst4.reshape(NC, NS, per_tile // CHUNK, CHUNK)

    degp = _sc_degree(dst4)
    deg2d = degp[:, :N].reshape(2 * N, 1)

    ycat, dinv = _tc_ka(x, fmask[None, :], W1, deg2d)
    acc1 = _sc_edge_pass(src4, dst4, ycat)
    acc1v = acc1[:, :N, :].reshape(2 * N, H)
    tcat = _tc_kb(acc1v, ycat, dinv, b1[None, :], W2)
    acc2 = _sc_edge_pass(src4, dst4, tcat)
    acc2v = acc2[:, :N, :].reshape(2 * N, H)
    batc = jnp.concatenate([batch, batch])[:, None]
    zcat = _tc_kc(acc2v, tcat, dinv, b2[None, :], batc)
    return (zcat[0], zcat[1])
